# baseline (speedup 1.0000x reference)
import functools

import jax
import jax.numpy as jnp
from jax import lax
from jax.experimental import pallas as pl
from jax.experimental.pallas import tpu as pltpu
from jax.experimental.pallas import tpu_sc as plsc


@functools.lru_cache(maxsize=None)
def _make_gather_kernel(B, D):
    info = plsc.get_sparse_core_info()
    NC, NS, L = info.num_cores, info.num_subcores, info.num_lanes
    NW = NC * NS
    b_per_w = B // NW
    assert B % (8 * NW) == 0 and D % L == 0
    mesh = plsc.VectorSubcoreMesh(core_axis_name="c", subcore_axis_name="s")

    @functools.partial(
        pl.kernel,
        mesh=mesh,
        compiler_params=pltpu.CompilerParams(use_tc_tiling_on_sc=False),
        out_type=(
            jax.ShapeDtypeStruct((B, D), jnp.float32),
            jax.ShapeDtypeStruct((B, D), jnp.float32),
        ),
        scratch_types=[
            pltpu.VMEM((b_per_w,), jnp.int32),
            pltpu.VMEM((b_per_w,), jnp.int32),
            pltpu.VMEM((b_per_w, D), jnp.float32),
            pltpu.VMEM((b_per_w, D), jnp.float32),
            pltpu.SemaphoreType.DMA,
            pltpu.SemaphoreType.DMA,
        ],
    )
    def k(uids_hbm, iids_hbm, ut_hbm, it_hbm, u_out, v_out,
          uidx_v, iidx_v, urows_v, irows_v, usem, isem):
        wid = lax.axis_index("s") * NC + lax.axis_index("c")
        base = wid * b_per_w
        pltpu.sync_copy(uids_hbm.at[pl.ds(base, b_per_w)], uidx_v)
        pltpu.sync_copy(iids_hbm.at[pl.ds(base, b_per_w)], iidx_v)
        ucopy = pltpu.async_copy(ut_hbm.at[uidx_v], urows_v, usem)
        icopy = pltpu.async_copy(it_hbm.at[iidx_v], irows_v, isem)
        ucopy.wait()
        pltpu.sync_copy(urows_v, u_out.at[pl.ds(base, b_per_w)])
        icopy.wait()
        pltpu.sync_copy(irows_v, v_out.at[pl.ds(base, b_per_w)])

    return k


def kernel(user_ids, item_ids, user_table, item_table):
    (B,) = user_ids.shape
    _, D = user_table.shape
    k = _make_gather_kernel(B, D)
    return k(user_ids.astype(jnp.int32), item_ids.astype(jnp.int32),
             user_table, item_table)


# speedup vs baseline: 1.4960x; 1.4960x over previous
import functools

import jax
import jax.numpy as jnp
from jax import lax
from jax.experimental import pallas as pl
from jax.experimental.pallas import tpu as pltpu
from jax.experimental.pallas import tpu_sc as plsc

_NSEM = 4


@functools.lru_cache(maxsize=None)
def _make_gather_kernel(B, D, rows_u, rows_i):
    info = plsc.get_sparse_core_info()
    NC, NS, L = info.num_cores, info.num_subcores, info.num_lanes
    NW = NC * NS
    b_per_w = B // NW
    assert B % (8 * NW) == 0 and D == 2 * L
    mesh = plsc.VectorSubcoreMesh(core_axis_name="c", subcore_axis_name="s")

    @functools.partial(
        pl.kernel,
        mesh=mesh,
        out_type=(
            jax.ShapeDtypeStruct((B, D), jnp.float32),
            jax.ShapeDtypeStruct((B, D), jnp.float32),
        ),
        scratch_types=[
            pltpu.VMEM((b_per_w,), jnp.int32),
            pltpu.VMEM((b_per_w, D), jnp.float32),
        ] + [pltpu.SemaphoreType.DMA] * _NSEM,
    )
    def k(uids_hbm, iids_hbm, ut_hbm, it_hbm, u_out, v_out,
          idx_v, rows_v, *sems):
        wid = lax.axis_index("s") * NC + lax.axis_index("c")
        base = wid * b_per_w

        for ids_hbm, tbl, o_hbm in ((uids_hbm, ut_hbm, u_out),
                                    (iids_hbm, it_hbm, v_out)):
            pltpu.sync_copy(ids_hbm.at[pl.ds(base, b_per_w)], idx_v)

            def group_body(g, _):
                idx16 = idx_v[pl.ds(g * L, L)]
                for l in range(L):
                    r = idx16[l]
                    pltpu.async_copy(
                        tbl.at[pl.ds(r, 1), :],
                        rows_v.at[pl.ds(g * L + l, 1), :],
                        sems[l % _NSEM],
                    )
                return 0
            lax.fori_loop(0, b_per_w // L, group_body, 0)
            for s in range(_NSEM):
                pltpu.make_async_copy(
                    tbl.at[pl.ds(0, b_per_w // _NSEM), :],
                    rows_v.at[pl.ds(0, b_per_w // _NSEM), :],
                    sems[s]).wait()
            pltpu.sync_copy(rows_v, o_hbm.at[pl.ds(base, b_per_w)])

    return k


def kernel(user_ids, item_ids, user_table, item_table):
    (B,) = user_ids.shape
    _, D = user_table.shape
    k = _make_gather_kernel(B, D, user_table.shape[0], item_table.shape[0])
    return k(user_ids.astype(jnp.int32), item_ids.astype(jnp.int32),
             user_table, item_table)


# speedup vs baseline: 1.4982x; 1.0015x over previous
import functools

import jax
import jax.numpy as jnp
from jax import lax
from jax.experimental import pallas as pl
from jax.experimental.pallas import tpu as pltpu
from jax.experimental.pallas import tpu_sc as plsc


@functools.lru_cache(maxsize=None)
def _make_gather_kernel(B, D, rows_u, rows_i):
    info = plsc.get_sparse_core_info()
    NC, NS, L = info.num_cores, info.num_subcores, info.num_lanes
    NW = NC * NS
    b_per_w = B // NW
    assert B % (8 * NW) == 0 and D == 2 * L
    mesh = plsc.VectorSubcoreMesh(core_axis_name="c", subcore_axis_name="s")

    @functools.partial(
        pl.kernel,
        mesh=mesh,
        out_type=(
            jax.ShapeDtypeStruct((B, D), jnp.float32),
            jax.ShapeDtypeStruct((B, D), jnp.float32),
        ),
        scratch_types=[
            pltpu.VMEM((b_per_w,), jnp.int32),
            pltpu.VMEM((b_per_w, D), jnp.float32),
            pltpu.SemaphoreType.DMA,
        ],
    )
    def k(uids_hbm, iids_hbm, ut_hbm, it_hbm, u_out, v_out,
          idx_v, rows_v, sem):
        wid = lax.axis_index("s") * NC + lax.axis_index("c")
        base = wid * b_per_w

        for ids_hbm, tbl, o_hbm in ((uids_hbm, ut_hbm, u_out),
                                    (iids_hbm, it_hbm, v_out)):
            pltpu.sync_copy(ids_hbm.at[pl.ds(base, b_per_w)], idx_v)

            def group_body(g, _):
                idx16 = idx_v[pl.ds(g * L, L)]
                for l in range(L):
                    r = idx16[l]
                    pltpu.async_copy(
                        tbl.at[pl.ds(r, 1), :],
                        rows_v.at[pl.ds(g * L + l, 1), :],
                        sem,
                    )
                return 0
            lax.fori_loop(0, b_per_w // L, group_body, 0)
            pltpu.make_async_copy(
                tbl.at[pl.ds(0, b_per_w), :], rows_v, sem).wait()
            pltpu.sync_copy(rows_v, o_hbm.at[pl.ds(base, b_per_w)])

    return k


def kernel(user_ids, item_ids, user_table, item_table):
    (B,) = user_ids.shape
    _, D = user_table.shape
    k = _make_gather_kernel(B, D, user_table.shape[0], item_table.shape[0])
    return k(user_ids.astype(jnp.int32), item_ids.astype(jnp.int32),
             user_table, item_table)
